# baseline (speedup 1.0000x reference)
import functools

import jax
import jax.numpy as jnp
from jax import lax
from jax.experimental import pallas as pl
from jax.experimental.pallas import tpu as pltpu
from jax.experimental.pallas import tpu_sc as plsc

_RING = 4
_NT = 16
_NSC = 2
_EPS = 1e-5


def _make_sc_segsum(w, passes, rng, gb, ept):
    mesh = plsc.VectorSubcoreMesh(core_axis_name="c", subcore_axis_name="s")
    nt = _NSC * _NT
    acc_rows = rng + 8
    sl = w // 128

    @functools.partial(
        pl.kernel,
        out_type=jax.ShapeDtypeStruct((passes * nt * rng, w), jnp.float32),
        mesh=mesh,
        scratch_types=[
            pltpu.VMEM((acc_rows, w), jnp.float32),
            pltpu.VMEM((ept,), jnp.int32),
            pltpu.VMEM((_RING, gb), jnp.int32),
            pltpu.VMEM((_RING, gb), jnp.int32),
            pltpu.VMEM((_RING, gb, sl, 128), jnp.float32),
            [pltpu.SemaphoreType.DMA] * _RING,
        ],
        compiler_params=pltpu.CompilerParams(needs_layout_passes=False),
    )
    def seg(table, packed, out, acc, pk_v, sg, sd, rows, sem_g):
        c = lax.axis_index("c")
        s = lax.axis_index("s")
        tq = s * _NSC + c

        def gather_start(b):
            pltpu.async_copy(table.at[sg.at[b]], rows.at[b], sem_g[b])

        def gather_wait(b):
            pltpu.make_async_copy(table.at[sg.at[0]], rows.at[b], sem_g[b]).wait()

        def one_pass(p, carry0):
            base = (p * nt + tq) * rng

            def zacc(i, carry):
                for j in range(w // 16):
                    acc[i, pl.ds(j * 16, 16)] = jnp.zeros((16,), jnp.float32)
                return carry
            lax.fori_loop(0, acc_rows, zacc, 0)

            def chunk_body(ch, cc):
                pltpu.sync_copy(packed.at[pl.ds(ch * ept, ept)], pk_v)

                def filt(gg, m):
                    pks, ks, cs = [], [], []
                    for u in range(8):
                        pk = pk_v[pl.ds(gg * 128 + u * 16, 16)]
                        d = lax.bitwise_and(pk, 16383) - base
                        k = lax.shift_right_logical(
                            lax.bitwise_and(lax.bitwise_not(d), d - rng), 31)
                        pks.append(pk)
                        ks.append(k)
                        cs.append(plsc.cumsum(k))
                    off = m
                    for u in range(8):
                        pos = ((off + cs[u] - 1) * ks[u]
                               + (gg * 128 + u * 16 + 15) * (1 - ks[u]))
                        plsc.store_scatter(pk_v, [pos], pks[u])
                        off = off + cs[u][15]
                    return off
                m = lax.fori_loop(0, ept // 128, filt, 0)
                ng = (m + gb - 1) // gb

                def prep(g, b):
                    for j in range(gb // 16):
                        idx0 = g * gb + j * 16
                        pk = pk_v[pl.ds(idx0, 16)]
                        v = lax.shift_right_logical(
                            idx0 + lax.iota(jnp.int32, 16) - m, 31)
                        srcv = lax.shift_right_logical(pk, 14)
                        d = lax.bitwise_and(pk, 16383) - base
                        sg[b, pl.ds(j * 16, 16)] = srcv * v
                        sd[b, pl.ds(j * 16, 16)] = d * v + rng * (1 - v)

                for g0 in range(_RING - 1):
                    @pl.when(g0 < ng)
                    def _(g0=g0):
                        prep(g0, g0)
                        gather_start(g0)

                def process_group(b):
                    def sub(j2, carry2):
                        dv = sd[b, pl.ds(j2 * 16, 16)]
                        rbase = j2 * 16
                        for r16 in range(16):
                            dloc = dv[r16]
                            for si in range(sl):
                                for j in range(8):
                                    plsc.addupdate(
                                        acc.at[dloc, pl.ds(si * 128 + j * 16, 16)],
                                        rows[b, rbase + r16, si, pl.ds(j * 16, 16)])
                        return carry2
                    lax.fori_loop(0, gb // 16, sub, 0)

                def step(i, carry):
                    for b in range(_RING):
                        g = i * _RING + b

                        @pl.when(g + _RING - 1 < ng)
                        def _():
                            prep(g + _RING - 1, (b + _RING - 1) % _RING)
                            gather_start((b + _RING - 1) % _RING)

                        @pl.when(g < ng)
                        def _():
                            gather_wait(b)
                            process_group(b)
                    return carry
                lax.fori_loop(0, (ng + _RING - 1) // _RING, step, 0)
                return cc
            lax.fori_loop(0, _NT, chunk_body, 0)

            pltpu.sync_copy(acc.at[pl.ds(0, rng)],
                            out.at[pl.ds(base, rng)])
            return carry0
        lax.fori_loop(0, passes, one_pass, 0)

    return seg


def _full(shape):
    return pl.BlockSpec(shape, lambda i: (0,) * len(shape))


def _k_combine_mm_stats(x_ref, agg_ref, w_ref, b_ref, h_ref, sum_ref, sq_ref):
    h = x_ref[...] + agg_ref[...]
    h_ref[...] = h
    z = jnp.dot(h, w_ref[...], preferred_element_type=jnp.float32) + b_ref[...]
    zs = jnp.sum(z, axis=0, keepdims=True)
    zq = jnp.sum(z * z, axis=0, keepdims=True)

    @pl.when(pl.program_id(0) == 0)
    def _():
        sum_ref[...] = zs
        sq_ref[...] = zq

    @pl.when(pl.program_id(0) != 0)
    def _():
        sum_ref[...] += zs
        sq_ref[...] += zq


def _combine_mm_stats(x, agg, w, b, bn):
    n, f_in = x.shape
    f_out = w.shape[1]
    grid = (n // bn,)
    return pl.pallas_call(
        _k_combine_mm_stats,
        grid=grid,
        in_specs=[
            pl.BlockSpec((bn, f_in), lambda i: (i, 0)),
            pl.BlockSpec((bn, f_in), lambda i: (i, 0)),
            _full(w.shape),
            _full((1, f_out)),
        ],
        out_specs=[
            pl.BlockSpec((bn, f_in), lambda i: (i, 0)),
            _full((1, f_out)),
            _full((1, f_out)),
        ],
        out_shape=[
            jax.ShapeDtypeStruct((n, f_in), jnp.float32),
            jax.ShapeDtypeStruct((1, f_out), jnp.float32),
            jax.ShapeDtypeStruct((1, f_out), jnp.float32),
        ],
    )(x, agg, w, b.reshape(1, f_out))


def _k_mlp_stats(h_ref, w1_ref, b1_ref, s1_ref, t1_ref, w2_ref, b2_ref,
                 u_ref, sum_ref, sq_ref):
    z = jnp.dot(h_ref[...], w1_ref[...], preferred_element_type=jnp.float32) + b1_ref[...]
    a = jnp.maximum(z * s1_ref[...] + t1_ref[...], 0.0)
    z2 = jnp.dot(a, w2_ref[...], preferred_element_type=jnp.float32) + b2_ref[...]
    u = jnp.maximum(z2, 0.0)
    u_ref[...] = u
    us = jnp.sum(u, axis=0, keepdims=True)
    uq = jnp.sum(u * u, axis=0, keepdims=True)

    @pl.when(pl.program_id(0) == 0)
    def _():
        sum_ref[...] = us
        sq_ref[...] = uq

    @pl.when(pl.program_id(0) != 0)
    def _():
        sum_ref[...] += us
        sq_ref[...] += uq


def _mlp_stats(h, w1, b1, s1, t1, w2, b2, bn):
    n, f_in = h.shape
    f_mid = w1.shape[1]
    f_out = w2.shape[1]
    grid = (n // bn,)
    return pl.pallas_call(
        _k_mlp_stats,
        grid=grid,
        in_specs=[
            pl.BlockSpec((bn, f_in), lambda i: (i, 0)),
            _full(w1.shape),
            _full((1, f_mid)),
            _full((1, f_mid)),
            _full((1, f_mid)),
            _full(w2.shape),
            _full((1, f_out)),
        ],
        out_specs=[
            pl.BlockSpec((bn, f_out), lambda i: (i, 0)),
            _full((1, f_out)),
            _full((1, f_out)),
        ],
        out_shape=[
            jax.ShapeDtypeStruct((n, f_out), jnp.float32),
            jax.ShapeDtypeStruct((1, f_out), jnp.float32),
            jax.ShapeDtypeStruct((1, f_out), jnp.float32),
        ],
    )(h, w1, b1.reshape(1, f_mid), s1.reshape(1, f_mid), t1.reshape(1, f_mid),
      w2, b2.reshape(1, f_out))


def _k_scale(u_ref, s_ref, t_ref, h_ref):
    h_ref[...] = u_ref[...] * s_ref[...] + t_ref[...]


def _scale(u, s, t, bn):
    n, f = u.shape
    return pl.pallas_call(
        _k_scale,
        grid=(n // bn,),
        in_specs=[
            pl.BlockSpec((bn, f), lambda i: (i, 0)),
            _full((1, f)),
            _full((1, f)),
        ],
        out_specs=pl.BlockSpec((bn, f), lambda i: (i, 0)),
        out_shape=jax.ShapeDtypeStruct((n, f), jnp.float32),
    )(u, s.reshape(1, f), t.reshape(1, f))


def _k_mlp_out(h2_ref, w1_ref, b1_ref, s1_ref, t1_ref, w2_ref, b2_ref,
               w3_ref, b3_ref, out_ref):
    z = jnp.dot(h2_ref[...], w1_ref[...], preferred_element_type=jnp.float32) + b1_ref[...]
    a = jnp.maximum(z * s1_ref[...] + t1_ref[...], 0.0)
    z2 = jnp.dot(a, w2_ref[...], preferred_element_type=jnp.float32) + b2_ref[...]
    v = jnp.maximum(z2, 0.0)
    out_ref[...] = jnp.dot(v, w3_ref[...], preferred_element_type=jnp.float32) + b3_ref[...]


def _mlp_out(h2, w1, b1, s1, t1, w2, b2, w3, b3, bn):
    n, f_in = h2.shape
    f_mid = w1.shape[1]
    f_mid2 = w2.shape[1]
    f_out = w3.shape[1]
    return pl.pallas_call(
        _k_mlp_out,
        grid=(n // bn,),
        in_specs=[
            pl.BlockSpec((bn, f_in), lambda i: (i, 0)),
            _full(w1.shape),
            _full((1, f_mid)),
            _full((1, f_mid)),
            _full((1, f_mid)),
            _full(w2.shape),
            _full((1, f_mid2)),
            _full(w3.shape),
            _full((1, f_out)),
        ],
        out_specs=pl.BlockSpec((bn, f_out), lambda i: (i, 0)),
        out_shape=jax.ShapeDtypeStruct((n, f_out), jnp.float32),
    )(h2, w1, b1.reshape(1, f_mid), s1.reshape(1, f_mid), t1.reshape(1, f_mid),
      w2, b2.reshape(1, f_mid2), w3, b3.reshape(1, f_out))


def _bn_scale_offset(ssum, ssq, g, b, n):
    m = ssum[0] / n
    v = ssq[0] / n - m * m
    s = g * lax.rsqrt(v + _EPS)
    t = b - m * s
    return s, t


def kernel(x, edge_index, W1a, b1a, g1a, be1a, W1b, b1b, go, bo,
           W2a, b2a, g2a, be2a, W2b, b2b, W3, b3):
    n = x.shape[0]
    src = edge_index[0].astype(jnp.int32)
    dst = edge_index[1].astype(jnp.int32)
    e = src.shape[0]

    ept = -(-e // (_NT * 128)) * 128
    e_pad = _NT * ept
    srcp = jnp.concatenate([src, jnp.zeros((e_pad - e,), jnp.int32)])
    dstp = jnp.concatenate([dst, jnp.full((e_pad - e,), 16383, jnp.int32)])
    packed = (srcp << 14) | dstp

    agg1 = _make_sc_segsum(256, 1, 320, 32, ept)(x.reshape(n, 2, 128), packed)
    h_in, zs, zq = _combine_mm_stats(x, agg1, W1a, b1a, 2000)
    s1, t1 = _bn_scale_offset(zs, zq, g1a, be1a, n)
    u, us, uq = _mlp_stats(h_in, W1a, b1a, s1, t1, W1b, b1b, 2000)
    so, to = _bn_scale_offset(us, uq, go, bo, n)
    h = _scale(u, so, to, 2000)

    agg2 = _make_sc_segsum(512, 2, 160, 16, ept)(h.reshape(n, 4, 128), packed)
    h2, zs2, zq2 = _combine_mm_stats(h, agg2, W2a, b2a, 2000)
    s2, t2 = _bn_scale_offset(zs2, zq2, g2a, be2a, n)
    out = _mlp_out(h2, W2a, b2a, s2, t2, W2b, b2b, W3, b3, 2000)
    return out


# speedup vs baseline: 3.7371x; 3.7371x over previous
import functools

import jax
import jax.numpy as jnp
from jax import lax
from jax.experimental import pallas as pl
from jax.experimental.pallas import tpu as pltpu
from jax.experimental.pallas import tpu_sc as plsc

_RING = 2
_NT = 16
_NSC = 2
_EPS = 1e-5


def _make_sc_segsum(w, passes, gb, ept):
    mesh = plsc.VectorSubcoreMesh(core_axis_name="c", subcore_axis_name="s")
    rng = 10240 // (passes * _NSC)
    acc_rows = rng + 128
    sl = w // 128
    rpt_z = acc_rows // _NT
    rpt_w = rng // _NT

    @functools.partial(
        pl.kernel,
        out_type=jax.ShapeDtypeStruct((passes * _NSC * rng, sl, 128),
                                      jnp.float32),
        mesh=mesh,
        scratch_types=[
            pltpu.VMEM_SHARED((acc_rows, sl, 128), jnp.float32),
            pltpu.VMEM((ept,), jnp.int32),
            pltpu.VMEM((_RING, gb), jnp.int32),
            pltpu.VMEM((_RING, gb), jnp.int32),
            pltpu.VMEM((_RING, gb, sl, 128), jnp.float32),
            [pltpu.SemaphoreType.DMA] * _RING,
        ],
        compiler_params=pltpu.CompilerParams(needs_layout_passes=False),
    )
    def seg(table, packed, out, acc, pk_v, sg, sd, rows, sem_g):
        c = lax.axis_index("c")
        s = lax.axis_index("s")

        def gather_start(b):
            pltpu.async_copy(table.at[sg.at[b]], rows.at[b], sem_g[b])

        def gather_wait(b):
            pltpu.make_async_copy(table.at[sg.at[0]], rows.at[b], sem_g[b]).wait()

        def one_pass(p, carry0):
            base = (p * _NSC + c) * rng
            plsc.subcore_barrier()

            def zb(i, carry):
                for si in range(sl):
                    for j in range(8):
                        rows[0, i, si, pl.ds(j * 16, 16)] = jnp.zeros(
                            (16,), jnp.float32)
                return carry
            lax.fori_loop(0, gb, zb, 0)

            def zacc(k, carry):
                pltpu.sync_copy(
                    rows.at[0].at[pl.ds(0, min(gb, rpt_z))],
                    acc.at[pl.ds(s * rpt_z + k * min(gb, rpt_z), min(gb, rpt_z))])
                return carry
            lax.fori_loop(0, rpt_z // min(gb, rpt_z), zacc, 0)
            tail = rpt_z - (rpt_z // min(gb, rpt_z)) * min(gb, rpt_z)
            if tail:
                pltpu.sync_copy(
                    rows.at[0].at[pl.ds(0, tail)],
                    acc.at[pl.ds(s * rpt_z + rpt_z - tail, tail)])

            pltpu.sync_copy(packed.at[pl.ds(s * ept, ept)], pk_v)

            def filt(gg, m):
                pks, ks, cs = [], [], []
                for u in range(8):
                    pk = pk_v[pl.ds(gg * 128 + u * 16, 16)]
                    d = lax.bitwise_and(pk, 16383) - base
                    k = lax.shift_right_logical(
                        lax.bitwise_and(lax.bitwise_not(d), d - rng), 31)
                    pks.append(pk)
                    ks.append(k)
                    cs.append(plsc.cumsum(k))
                off = m
                for u in range(8):
                    pos = ((off + cs[u] - 1) * ks[u]
                           + (gg * 128 + u * 16 + 15) * (1 - ks[u]))
                    plsc.store_scatter(pk_v, [pos], pks[u])
                    off = off + cs[u][15]
                return off
            m = lax.fori_loop(0, ept // 128, filt, 0)
            ng = (m + gb - 1) // gb

            def prep(g, b):
                for j in range(gb // 16):
                    idx0 = g * gb + j * 16
                    pk = pk_v[pl.ds(idx0, 16)]
                    v = lax.shift_right_logical(
                        idx0 + lax.iota(jnp.int32, 16) - m, 31)
                    srcv = lax.shift_right_logical(pk, 14)
                    d = lax.bitwise_and(pk, 16383) - base
                    sg[b, pl.ds(j * 16, 16)] = srcv * v
                    sd[b, pl.ds(j * 16, 16)] = d * v + rng * (1 - v)

            plsc.subcore_barrier()

            @pl.when(0 < ng)
            def _():
                prep(0, 0)
                gather_start(0)

            def step(i, carry):
                for b in range(2):
                    g = i * 2 + b

                    @pl.when(g + 1 < ng)
                    def _():
                        prep(g + 1, 1 - b)
                        gather_start(1 - b)

                    @pl.when(g < ng)
                    def _():
                        gather_wait(b)
                        pltpu.sync_copy(rows.at[b], acc.at[sd.at[b]], add=True)
                return carry
            lax.fori_loop(0, (ng + 1) // 2, step, 0)
            plsc.subcore_barrier()

            def wout(k, carry):
                ww = min(gb, rpt_w)
                off = s * rpt_w + k * ww
                pltpu.sync_copy(acc.at[pl.ds(off, ww)], rows.at[0])
                pltpu.sync_copy(rows.at[0], out.at[pl.ds(base + off, ww)])
                return carry
            lax.fori_loop(0, rpt_w // min(gb, rpt_w), wout, 0)
            return carry0
        lax.fori_loop(0, passes, one_pass, 0)

    return seg


def _full(shape):
    return pl.BlockSpec(shape, lambda i: (0,) * len(shape))


def _k_combine_mm_stats(x_ref, agg_ref, w_ref, b_ref, h_ref, sum_ref, sq_ref):
    h = x_ref[...] + agg_ref[...]
    h_ref[...] = h
    z = jnp.dot(h, w_ref[...], preferred_element_type=jnp.float32) + b_ref[...]
    zs = jnp.sum(z, axis=0, keepdims=True)
    zq = jnp.sum(z * z, axis=0, keepdims=True)

    @pl.when(pl.program_id(0) == 0)
    def _():
        sum_ref[...] = zs
        sq_ref[...] = zq

    @pl.when(pl.program_id(0) != 0)
    def _():
        sum_ref[...] += zs
        sq_ref[...] += zq


def _combine_mm_stats(x, agg, w, b, bn):
    n, f_in = x.shape
    f_out = w.shape[1]
    grid = (n // bn,)
    return pl.pallas_call(
        _k_combine_mm_stats,
        grid=grid,
        in_specs=[
            pl.BlockSpec((bn, f_in), lambda i: (i, 0)),
            pl.BlockSpec((bn, f_in), lambda i: (i, 0)),
            _full(w.shape),
            _full((1, f_out)),
        ],
        out_specs=[
            pl.BlockSpec((bn, f_in), lambda i: (i, 0)),
            _full((1, f_out)),
            _full((1, f_out)),
        ],
        out_shape=[
            jax.ShapeDtypeStruct((n, f_in), jnp.float32),
            jax.ShapeDtypeStruct((1, f_out), jnp.float32),
            jax.ShapeDtypeStruct((1, f_out), jnp.float32),
        ],
    )(x, agg, w, b.reshape(1, f_out))


def _k_mlp_stats(h_ref, w1_ref, b1_ref, s1_ref, t1_ref, w2_ref, b2_ref,
                 u_ref, sum_ref, sq_ref):
    z = jnp.dot(h_ref[...], w1_ref[...], preferred_element_type=jnp.float32) + b1_ref[...]
    a = jnp.maximum(z * s1_ref[...] + t1_ref[...], 0.0)
    z2 = jnp.dot(a, w2_ref[...], preferred_element_type=jnp.float32) + b2_ref[...]
    u = jnp.maximum(z2, 0.0)
    u_ref[...] = u
    us = jnp.sum(u, axis=0, keepdims=True)
    uq = jnp.sum(u * u, axis=0, keepdims=True)

    @pl.when(pl.program_id(0) == 0)
    def _():
        sum_ref[...] = us
        sq_ref[...] = uq

    @pl.when(pl.program_id(0) != 0)
    def _():
        sum_ref[...] += us
        sq_ref[...] += uq


def _mlp_stats(h, w1, b1, s1, t1, w2, b2, bn):
    n, f_in = h.shape
    f_mid = w1.shape[1]
    f_out = w2.shape[1]
    grid = (n // bn,)
    return pl.pallas_call(
        _k_mlp_stats,
        grid=grid,
        in_specs=[
            pl.BlockSpec((bn, f_in), lambda i: (i, 0)),
            _full(w1.shape),
            _full((1, f_mid)),
            _full((1, f_mid)),
            _full((1, f_mid)),
            _full(w2.shape),
            _full((1, f_out)),
        ],
        out_specs=[
            pl.BlockSpec((bn, f_out), lambda i: (i, 0)),
            _full((1, f_out)),
            _full((1, f_out)),
        ],
        out_shape=[
            jax.ShapeDtypeStruct((n, f_out), jnp.float32),
            jax.ShapeDtypeStruct((1, f_out), jnp.float32),
            jax.ShapeDtypeStruct((1, f_out), jnp.float32),
        ],
    )(h, w1, b1.reshape(1, f_mid), s1.reshape(1, f_mid), t1.reshape(1, f_mid),
      w2, b2.reshape(1, f_out))


def _k_scale(u_ref, s_ref, t_ref, h_ref):
    h_ref[...] = u_ref[...] * s_ref[...] + t_ref[...]


def _scale(u, s, t, bn):
    n, f = u.shape
    return pl.pallas_call(
        _k_scale,
        grid=(n // bn,),
        in_specs=[
            pl.BlockSpec((bn, f), lambda i: (i, 0)),
            _full((1, f)),
            _full((1, f)),
        ],
        out_specs=pl.BlockSpec((bn, f), lambda i: (i, 0)),
        out_shape=jax.ShapeDtypeStruct((n, f), jnp.float32),
    )(u, s.reshape(1, f), t.reshape(1, f))


def _k_mlp_out(h2_ref, w1_ref, b1_ref, s1_ref, t1_ref, w2_ref, b2_ref,
               w3_ref, b3_ref, out_ref):
    z = jnp.dot(h2_ref[...], w1_ref[...], preferred_element_type=jnp.float32) + b1_ref[...]
    a = jnp.maximum(z * s1_ref[...] + t1_ref[...], 0.0)
    z2 = jnp.dot(a, w2_ref[...], preferred_element_type=jnp.float32) + b2_ref[...]
    v = jnp.maximum(z2, 0.0)
    out_ref[...] = jnp.dot(v, w3_ref[...], preferred_element_type=jnp.float32) + b3_ref[...]


def _mlp_out(h2, w1, b1, s1, t1, w2, b2, w3, b3, bn):
    n, f_in = h2.shape
    f_mid = w1.shape[1]
    f_mid2 = w2.shape[1]
    f_out = w3.shape[1]
    return pl.pallas_call(
        _k_mlp_out,
        grid=(n // bn,),
        in_specs=[
            pl.BlockSpec((bn, f_in), lambda i: (i, 0)),
            _full(w1.shape),
            _full((1, f_mid)),
            _full((1, f_mid)),
            _full((1, f_mid)),
            _full(w2.shape),
            _full((1, f_mid2)),
            _full(w3.shape),
            _full((1, f_out)),
        ],
        out_specs=pl.BlockSpec((bn, f_out), lambda i: (i, 0)),
        out_shape=jax.ShapeDtypeStruct((n, f_out), jnp.float32),
    )(h2, w1, b1.reshape(1, f_mid), s1.reshape(1, f_mid), t1.reshape(1, f_mid),
      w2, b2.reshape(1, f_mid2), w3, b3.reshape(1, f_out))


def _bn_scale_offset(ssum, ssq, g, b, n):
    m = ssum[0] / n
    v = ssq[0] / n - m * m
    s = g * lax.rsqrt(v + _EPS)
    t = b - m * s
    return s, t


def kernel(x, edge_index, W1a, b1a, g1a, be1a, W1b, b1b, go, bo,
           W2a, b2a, g2a, be2a, W2b, b2b, W3, b3):
    n = x.shape[0]
    src = edge_index[0].astype(jnp.int32)
    dst = edge_index[1].astype(jnp.int32)
    e = src.shape[0]

    ept = -(-e // (_NT * 128)) * 128
    e_pad = _NT * ept
    srcp = jnp.concatenate([src, jnp.zeros((e_pad - e,), jnp.int32)])
    dstp = jnp.concatenate([dst, jnp.full((e_pad - e,), 16383, jnp.int32)])
    packed = (srcp << 14) | dstp

    agg1 = _make_sc_segsum(256, 1, 64, ept)(x.reshape(n, 2, 128), packed)
    h_in, zs, zq = _combine_mm_stats(x, agg1.reshape(-1, 256), W1a, b1a, 2000)
    s1, t1 = _bn_scale_offset(zs, zq, g1a, be1a, n)
    u, us, uq = _mlp_stats(h_in, W1a, b1a, s1, t1, W1b, b1b, 2000)
    so, to = _bn_scale_offset(us, uq, go, bo, n)
    h = _scale(u, so, to, 2000)

    agg2 = _make_sc_segsum(512, 2, 32, ept)(h.reshape(n, 4, 128), packed)
    h2, zs2, zq2 = _combine_mm_stats(h, agg2.reshape(-1, 512), W2a, b2a, 2000)
    s2, t2 = _bn_scale_offset(zs2, zq2, g2a, be2a, n)
    out = _mlp_out(h2, W2a, b2a, s2, t2, W2b, b2b, W3, b3, 2000)
    return out
